# Initial kernel scaffold; baseline (speedup 1.0000x reference)
import functools

import jax
import jax.numpy as jnp
from jax import lax
from jax.experimental import pallas as pl
from jax.experimental.pallas import tpu as pltpu
from jax.experimental.pallas import tpu_sc as plsc

N = 10000
E = 320000
D = 128
NC = 2
NS = 16
NW = NC * NS
EPW = E // NW
CHUNK = 125
NCHUNK = EPW // CHUNK
RPT = N // NS
DEGW = 8

_BN_SCALE = 1.0 / float(jnp.sqrt(jnp.float32(1.0 + 1e-5)))


def _edge_pass_body(y_hbm, src_hbm, dst_hbm, zsum_hbm, zdeg_hbm, ones_hbm,
                    sum_out, deg_out,
                    acc_sh, deg_sh, src_v, dst_v, rows0, rows1, ones_v,
                    sem0, sem1):
    c = lax.axis_index("c")
    s = lax.axis_index("s")
    wid = s * NC + c

    r = pl.ds(s * RPT, RPT)
    pltpu.sync_copy(zsum_hbm.at[r], acc_sh.at[r])
    pltpu.sync_copy(zdeg_hbm.at[r], deg_sh.at[r])

    pltpu.sync_copy(src_hbm.at[wid], src_v)
    pltpu.sync_copy(dst_hbm.at[wid], dst_v)
    pltpu.sync_copy(ones_hbm, ones_v)
    plsc.subcore_barrier()

    rows = (rows0, rows1)
    sems = (sem0, sem1)
    for b in range(2):
        pltpu.async_copy(y_hbm.at[src_v.at[b]], rows[b], sems[b])

    def body(it, carry):
        j = it * 2
        for b in range(2):
            jb = j + b
            pltpu.make_async_copy(y_hbm.at[src_v.at[jb]], rows[b],
                                  sems[b]).wait()
            pltpu.sync_copy(rows[b], acc_sh.at[dst_v.at[jb]], add=True)
            pltpu.sync_copy(ones_v, deg_sh.at[dst_v.at[jb]], add=True)
            nxt = jb + 2

            @pl.when(nxt < NCHUNK)
            def _():
                pltpu.async_copy(y_hbm.at[src_v.at[nxt]], rows[b], sems[b])
        return carry

    lax.fori_loop(0, NCHUNK // 2, body, 0)

    plsc.subcore_barrier()
    pltpu.sync_copy(acc_sh.at[r], sum_out.at[c, r])
    pltpu.sync_copy(deg_sh.at[r], deg_out.at[c, r])


_edge_pass = pl.kernel(
    _edge_pass_body,
    out_type=[jax.ShapeDtypeStruct((NC, N, D), jnp.float32),
              jax.ShapeDtypeStruct((NC, N, DEGW), jnp.float32)],
    mesh=plsc.VectorSubcoreMesh(core_axis_name="c", subcore_axis_name="s"),
    scratch_types=[
        pltpu.VMEM_SHARED((N, D), jnp.float32),
        pltpu.VMEM_SHARED((N, DEGW), jnp.float32),
        pltpu.VMEM((NCHUNK, CHUNK), jnp.int32),
        pltpu.VMEM((NCHUNK, CHUNK), jnp.int32),
        pltpu.VMEM((CHUNK, D), jnp.float32),
        pltpu.VMEM((CHUNK, D), jnp.float32),
        pltpu.VMEM((CHUNK, DEGW), jnp.float32),
        pltpu.SemaphoreType.DMA,
        pltpu.SemaphoreType.DMA,
    ],
)


BLK = 1000


def _pre_body(x_ref, w_ref, b_ref, o_ref):
    o_ref[...] = jnp.maximum(
        jnp.dot(x_ref[...], w_ref[...], preferred_element_type=jnp.float32)
        + b_ref[...], 0.0)


_pre = pl.pallas_call(
    _pre_body,
    grid=(N // BLK,),
    in_specs=[pl.BlockSpec((BLK, D), lambda i: (i, 0)),
              pl.BlockSpec((D, D), lambda i: (0, 0)),
              pl.BlockSpec((1, D), lambda i: (0, 0))],
    out_specs=pl.BlockSpec((BLK, D), lambda i: (i, 0)),
    out_shape=jax.ShapeDtypeStruct((N, D), jnp.float32),
)


def _agg_from_parts(s_ref, d_ref):
    summed = s_ref[0] + s_ref[1]
    deg = d_ref[0, :, 0:1] + d_ref[1, :, 0:1]
    return summed * (1.0 / jnp.maximum(deg, 1.0))


def _combine_body(y_ref, s_ref, d_ref, ws_ref, wn_ref, b_ref, g_ref, be_ref,
                  o_ref):
    agg = _agg_from_parts(s_ref, d_ref)
    h = (jnp.dot(y_ref[...], ws_ref[...], preferred_element_type=jnp.float32)
         + jnp.dot(agg, wn_ref[...], preferred_element_type=jnp.float32)
         + b_ref[...])
    h = h * (_BN_SCALE * g_ref[...]) + be_ref[...]
    o_ref[...] = jnp.maximum(h, 0.0)


_combine = pl.pallas_call(
    _combine_body,
    grid=(N // BLK,),
    in_specs=[pl.BlockSpec((BLK, D), lambda i: (i, 0)),
              pl.BlockSpec((NC, BLK, D), lambda i: (0, i, 0)),
              pl.BlockSpec((NC, BLK, DEGW), lambda i: (0, i, 0)),
              pl.BlockSpec((D, D), lambda i: (0, 0)),
              pl.BlockSpec((D, D), lambda i: (0, 0)),
              pl.BlockSpec((1, D), lambda i: (0, 0)),
              pl.BlockSpec((1, D), lambda i: (0, 0)),
              pl.BlockSpec((1, D), lambda i: (0, 0))],
    out_specs=pl.BlockSpec((BLK, D), lambda i: (i, 0)),
    out_shape=jax.ShapeDtypeStruct((N, D), jnp.float32),
)


def _final_body(y_ref, s_ref, d_ref, ws_ref, wn_ref, b_ref, g_ref, be_ref,
                w2_ref, b2_ref, o_ref):
    agg = _agg_from_parts(s_ref, d_ref)
    h = (jnp.dot(y_ref[...], ws_ref[...], preferred_element_type=jnp.float32)
         + jnp.dot(agg, wn_ref[...], preferred_element_type=jnp.float32)
         + b_ref[...])
    h = h * (_BN_SCALE * g_ref[...]) + be_ref[...]
    o_ref[...] = (jnp.dot(h, w2_ref[...], preferred_element_type=jnp.float32)
                  + b2_ref[...])


_final = pl.pallas_call(
    _final_body,
    grid=(N // BLK,),
    in_specs=[pl.BlockSpec((BLK, D), lambda i: (i, 0)),
              pl.BlockSpec((NC, BLK, D), lambda i: (0, i, 0)),
              pl.BlockSpec((NC, BLK, DEGW), lambda i: (0, i, 0)),
              pl.BlockSpec((D, D), lambda i: (0, 0)),
              pl.BlockSpec((D, D), lambda i: (0, 0)),
              pl.BlockSpec((1, D), lambda i: (0, 0)),
              pl.BlockSpec((1, D), lambda i: (0, 0)),
              pl.BlockSpec((1, D), lambda i: (0, 0)),
              pl.BlockSpec((D, 1), lambda i: (0, 0)),
              pl.BlockSpec((1, 1), lambda i: (0, 0))],
    out_specs=pl.BlockSpec((BLK, 1), lambda i: (i, 0)),
    out_shape=jax.ShapeDtypeStruct((N, 1), jnp.float32),
)


def kernel(x, edge_index, W1, b1, Wself0, Wneigh0, bconv0, gamma0, beta0,
           Wself1, Wneigh1, bconv1, gamma1, beta1, W2, b2):
    src3 = edge_index[0].reshape(NW, NCHUNK, CHUNK)
    dst3 = edge_index[1].reshape(NW, NCHUNK, CHUNK)
    zsum = jnp.zeros((N, D), jnp.float32)
    zdeg = jnp.zeros((N, DEGW), jnp.float32)
    ones = jnp.ones((CHUNK, DEGW), jnp.float32)

    y0 = _pre(x, W1, b1.reshape(1, D))
    s0, d0 = _edge_pass(y0, src3, dst3, zsum, zdeg, ones)
    y1 = _combine(y0, s0, d0, Wself0, Wneigh0, bconv0.reshape(1, D),
                  gamma0.reshape(1, D), beta0.reshape(1, D))
    s1, d1 = _edge_pass(y1, src3, dst3, zsum, zdeg, ones)
    out = _final(y1, s1, d1, Wself1, Wneigh1, bconv1.reshape(1, D),
                 gamma1.reshape(1, D), beta1.reshape(1, D), W2,
                 b2.reshape(1, 1))
    return out


# speedup vs baseline: 12.5143x; 12.5143x over previous
import math

import jax
import jax.numpy as jnp
from jax import lax
from jax.experimental import pallas as pl
from jax.experimental.pallas import tpu as pltpu
from jax.experimental.pallas import tpu_sc as plsc

N = 10000
E = 320000
D = 128
NC = 2
NS = 16
NW = NC * NS
EPW = E // NW
CHUNK = 100
NCHUNK = EPW // CHUNK
RPT = 632
NPAD = NS * RPT
DEGW = 8

_BN_SCALE = 1.0 / math.sqrt(1.0 + 1e-5)


def _deg_pass_body(dst_hbm, zdeg_hbm, ones_hbm, deg_out,
                   deg_sh, dst_v, ones_v, sem0, sem1):
    c = lax.axis_index("c")
    s = lax.axis_index("s")
    wid = s * NC + c

    r = pl.ds(s * RPT, RPT)
    pltpu.sync_copy(zdeg_hbm.at[r], deg_sh.at[r])
    pltpu.sync_copy(dst_hbm.at[wid], dst_v)
    pltpu.sync_copy(ones_hbm, ones_v)
    plsc.subcore_barrier()

    sems = (sem0, sem1)
    for b in range(2):
        pltpu.async_copy(ones_v, deg_sh.at[dst_v.at[b]], sems[b], add=True)

    def body(it, carry):
        j = it * 2
        for b in range(2):
            jb = j + b
            pltpu.make_async_copy(ones_v, deg_sh.at[dst_v.at[jb]],
                                  sems[b]).wait()
            nxt = jb + 2

            @pl.when(nxt < NCHUNK)
            def _():
                pltpu.async_copy(ones_v, deg_sh.at[dst_v.at[nxt]], sems[b],
                                 add=True)
        return carry

    lax.fori_loop(0, NCHUNK // 2, body, 0)

    plsc.subcore_barrier()
    pltpu.sync_copy(deg_sh.at[r], deg_out.at[c, r])


_deg_pass = pl.kernel(
    _deg_pass_body,
    out_type=[jax.ShapeDtypeStruct((NC, NPAD, DEGW), jnp.float32)],
    mesh=plsc.VectorSubcoreMesh(core_axis_name="c", subcore_axis_name="s"),
    compiler_params=pltpu.CompilerParams(use_tc_tiling_on_sc=False),
    scratch_types=[
        pltpu.VMEM_SHARED((NPAD, DEGW), jnp.float32),
        pltpu.VMEM((NCHUNK, CHUNK), jnp.int32),
        pltpu.VMEM((CHUNK, DEGW), jnp.float32),
        pltpu.SemaphoreType.DMA,
        pltpu.SemaphoreType.DMA,
    ],
)


def _edge_pass_body(y_hbm, src_hbm, dst_hbm, zsum_hbm,
                    sum_out,
                    acc_sh, src_v, dst_v, rows0, rows1, sem0, sem1):
    c = lax.axis_index("c")
    s = lax.axis_index("s")
    wid = s * NC + c

    r = pl.ds(s * RPT, RPT)
    pltpu.sync_copy(zsum_hbm.at[r], acc_sh.at[r])

    pltpu.sync_copy(src_hbm.at[wid], src_v)
    pltpu.sync_copy(dst_hbm.at[wid], dst_v)
    plsc.subcore_barrier()

    rows = (rows0, rows1)
    sems = (sem0, sem1)
    for b in range(2):
        pltpu.async_copy(y_hbm.at[src_v.at[b]], rows[b], sems[b])

    def body(it, carry):
        j = it * 2
        for b in range(2):
            jb = j + b
            pltpu.make_async_copy(y_hbm.at[src_v.at[jb]], rows[b],
                                  sems[b]).wait()
            pltpu.sync_copy(rows[b], acc_sh.at[dst_v.at[jb]], add=True)
            nxt = jb + 2

            @pl.when(nxt < NCHUNK)
            def _():
                pltpu.async_copy(y_hbm.at[src_v.at[nxt]], rows[b], sems[b])
        return carry

    lax.fori_loop(0, NCHUNK // 2, body, 0)

    plsc.subcore_barrier()
    pltpu.sync_copy(acc_sh.at[r], sum_out.at[c, r])


_edge_pass = pl.kernel(
    _edge_pass_body,
    out_type=[jax.ShapeDtypeStruct((NC, NPAD, D), jnp.float32)],
    mesh=plsc.VectorSubcoreMesh(core_axis_name="c", subcore_axis_name="s"),
    compiler_params=pltpu.CompilerParams(use_tc_tiling_on_sc=False),
    scratch_types=[
        pltpu.VMEM_SHARED((NPAD, D), jnp.float32),
        pltpu.VMEM((NCHUNK, CHUNK), jnp.int32),
        pltpu.VMEM((NCHUNK, CHUNK), jnp.int32),
        pltpu.VMEM((CHUNK, D), jnp.float32),
        pltpu.VMEM((CHUNK, D), jnp.float32),
        pltpu.SemaphoreType.DMA,
        pltpu.SemaphoreType.DMA,
    ],
)


BLK = 1000


def _pre_body(x_ref, w_ref, b_ref, o_ref):
    o_ref[...] = jnp.maximum(
        jnp.dot(x_ref[...], w_ref[...], preferred_element_type=jnp.float32)
        + b_ref[...], 0.0)


_pre = pl.pallas_call(
    _pre_body,
    grid=(N // BLK,),
    in_specs=[pl.BlockSpec((BLK, D), lambda i: (i, 0)),
              pl.BlockSpec((D, D), lambda i: (0, 0)),
              pl.BlockSpec((1, D), lambda i: (0, 0))],
    out_specs=pl.BlockSpec((BLK, D), lambda i: (i, 0)),
    out_shape=jax.ShapeDtypeStruct((N, D), jnp.float32),
)


def _agg_from_parts(s_ref, d_ref):
    summed = s_ref[0] + s_ref[1]
    deg = d_ref[0, :, 0:1] + d_ref[1, :, 0:1]
    return summed * (1.0 / jnp.maximum(deg, 1.0))


def _combine_body(y_ref, s_ref, d_ref, ws_ref, wn_ref, b_ref, g_ref, be_ref,
                  o_ref):
    agg = _agg_from_parts(s_ref, d_ref)
    h = (jnp.dot(y_ref[...], ws_ref[...], preferred_element_type=jnp.float32)
         + jnp.dot(agg, wn_ref[...], preferred_element_type=jnp.float32)
         + b_ref[...])
    h = h * (_BN_SCALE * g_ref[...]) + be_ref[...]
    o_ref[...] = jnp.maximum(h, 0.0)


_combine = pl.pallas_call(
    _combine_body,
    grid=(N // BLK,),
    in_specs=[pl.BlockSpec((BLK, D), lambda i: (i, 0)),
              pl.BlockSpec((NC, BLK, D), lambda i: (0, i, 0)),
              pl.BlockSpec((NC, BLK, DEGW), lambda i: (0, i, 0)),
              pl.BlockSpec((D, D), lambda i: (0, 0)),
              pl.BlockSpec((D, D), lambda i: (0, 0)),
              pl.BlockSpec((1, D), lambda i: (0, 0)),
              pl.BlockSpec((1, D), lambda i: (0, 0)),
              pl.BlockSpec((1, D), lambda i: (0, 0))],
    out_specs=pl.BlockSpec((BLK, D), lambda i: (i, 0)),
    out_shape=jax.ShapeDtypeStruct((N, D), jnp.float32),
)


def _final_body(y_ref, s_ref, d_ref, ws_ref, wn_ref, b_ref, g_ref, be_ref,
                w2_ref, b2_ref, o_ref):
    agg = _agg_from_parts(s_ref, d_ref)
    h = (jnp.dot(y_ref[...], ws_ref[...], preferred_element_type=jnp.float32)
         + jnp.dot(agg, wn_ref[...], preferred_element_type=jnp.float32)
         + b_ref[...])
    h = h * (_BN_SCALE * g_ref[...]) + be_ref[...]
    o_ref[...] = (jnp.dot(h, w2_ref[...], preferred_element_type=jnp.float32)
                  + b2_ref[...])


_final = pl.pallas_call(
    _final_body,
    grid=(N // BLK,),
    in_specs=[pl.BlockSpec((BLK, D), lambda i: (i, 0)),
              pl.BlockSpec((NC, BLK, D), lambda i: (0, i, 0)),
              pl.BlockSpec((NC, BLK, DEGW), lambda i: (0, i, 0)),
              pl.BlockSpec((D, D), lambda i: (0, 0)),
              pl.BlockSpec((D, D), lambda i: (0, 0)),
              pl.BlockSpec((1, D), lambda i: (0, 0)),
              pl.BlockSpec((1, D), lambda i: (0, 0)),
              pl.BlockSpec((1, D), lambda i: (0, 0)),
              pl.BlockSpec((D, 1), lambda i: (0, 0)),
              pl.BlockSpec((1, 1), lambda i: (0, 0))],
    out_specs=pl.BlockSpec((BLK, 1), lambda i: (i, 0)),
    out_shape=jax.ShapeDtypeStruct((N, 1), jnp.float32),
)


def kernel(x, edge_index, W1, b1, Wself0, Wneigh0, bconv0, gamma0, beta0,
           Wself1, Wneigh1, bconv1, gamma1, beta1, W2, b2):
    src3 = edge_index[0].reshape(NW, NCHUNK, CHUNK)
    dst3 = edge_index[1].reshape(NW, NCHUNK, CHUNK)
    zsum = jnp.zeros((NPAD, D), jnp.float32)
    zdeg = jnp.zeros((NPAD, DEGW), jnp.float32)
    ones = jnp.ones((CHUNK, DEGW), jnp.float32)

    (deg,) = _deg_pass(dst3, zdeg, ones)
    y0 = _pre(x, W1, b1.reshape(1, D))
    (s0,) = _edge_pass(y0, src3, dst3, zsum)
    y1 = _combine(y0, s0, deg, Wself0, Wneigh0, bconv0.reshape(1, D),
                  gamma0.reshape(1, D), beta0.reshape(1, D))
    (s1,) = _edge_pass(y1, src3, dst3, zsum)
    out = _final(y1, s1, deg, Wself1, Wneigh1, bconv1.reshape(1, D),
                 gamma1.reshape(1, D), beta1.reshape(1, D), W2,
                 b2.reshape(1, 1))
    return out
